# baseline (speedup 1.0000x reference)
import functools

import jax
import jax.numpy as jnp
from jax import lax
from jax.experimental import pallas as pl
from jax.experimental.pallas import tpu as pltpu
from jax.experimental.pallas import tpu_sc as plsc

N = 10000
E = 320000
D = 128
G = 128
NCLS = 10
NC, NS = 2, 16
NW = NC * NS
EPW = E // NW
CHUNK = 128
NB = 2
NG = 40
CPW = NB * NG
PAD = CPW * CHUNK - EPW
ACC_R = N + 8
RPT = 624
TAIL = N - NS * RPT
DEGW = 16
RB = 1000

def _mesh():
    return plsc.VectorSubcoreMesh(core_axis_name="c", subcore_axis_name="s",
                                  num_cores=NC, num_subcores=NS)


def _sc_degree(ones_hbm, dst_w):
    @functools.partial(
        pl.kernel,
        out_type=jax.ShapeDtypeStruct((NC, N, DEGW), jnp.float32),
        mesh=_mesh(),
        scratch_types=[
            pltpu.VMEM((CPW, CHUNK), jnp.int32),
            pltpu.VMEM((CHUNK, DEGW), jnp.float32),
            pltpu.SemaphoreType.DMA,
            pltpu.VMEM_SHARED((ACC_R, DEGW), jnp.float32),
        ],
    )
    def k(ones_ref, dst_ref, out_ref, idx_v, ones_v, ssem, acc):
        cid = lax.axis_index("c")
        sid = lax.axis_index("s")
        w = cid * NS + sid
        pltpu.sync_copy(dst_ref.at[w], idx_v)
        pltpu.sync_copy(ones_ref.at[pl.ds(0, CHUNK)], ones_v)
        rs = pl.ds(sid * RPT, RPT)
        ts = pl.ds(NS * RPT, TAIL)
        pltpu.sync_copy(ones_ref.at[pl.ds(0, RPT)], acc.at[rs])

        @pl.when(sid == NS - 1)
        def _():
            pltpu.sync_copy(ones_ref.at[pl.ds(0, TAIL)], acc.at[ts])

        plsc.subcore_barrier()

        @pl.loop(0, CPW)
        def _(j):
            pltpu.async_copy(ones_v, acc.at[idx_v.at[j]], ssem, add=True)

        @pl.loop(0, CPW)
        def _(j):
            pltpu.make_async_copy(ones_v, acc.at[idx_v.at[j]], ssem).wait()

        plsc.subcore_barrier()
        pltpu.sync_copy(acc.at[rs], out_ref.at[cid, rs])

        @pl.when(sid == NS - 1)
        def _():
            pltpu.sync_copy(acc.at[ts], out_ref.at[cid, ts])

    return k(ones_hbm, dst_w)


def _sc_aggregate(t, src_w, dst_w):
    @functools.partial(
        pl.kernel,
        out_type=jax.ShapeDtypeStruct((NC, N, D), jnp.float32),
        mesh=_mesh(),
        scratch_types=[
            pltpu.VMEM((CPW, CHUNK), jnp.int32),
            pltpu.VMEM((CPW, CHUNK), jnp.int32),
            pltpu.VMEM((CHUNK, D), jnp.float32),
            pltpu.VMEM_SHARED((ACC_R, D), jnp.float32),
        ],
    )
    def k(t_ref, src_ref, dst_ref, out_ref, srcv, didx, rowbuf, acc):
        cid = lax.axis_index("c")
        sid = lax.axis_index("s")
        w = cid * NS + sid
        pltpu.sync_copy(src_ref.at[w], srcv)
        pltpu.sync_copy(dst_ref.at[w], didx)
        rs = pl.ds(sid * RPT, RPT)
        ts = pl.ds(NS * RPT, TAIL)
        pltpu.sync_copy(t_ref.at[rs], acc.at[rs])

        @pl.when(sid == NS - 1)
        def _():
            pltpu.sync_copy(t_ref.at[ts], acc.at[ts])

        plsc.subcore_barrier()

        @pl.loop(0, CPW)
        def _(j):
            pltpu.sync_copy(t_ref.at[srcv.at[j]], rowbuf)
            pltpu.sync_copy(rowbuf, acc.at[didx.at[j]], add=True)

        plsc.subcore_barrier()
        pltpu.sync_copy(acc.at[rs], out_ref.at[cid, rs])

        @pl.when(sid == NS - 1)
        def _():
            pltpu.sync_copy(acc.at[ts], out_ref.at[cid, ts])

    return k(t, src_w, dst_w)


def _elu(v):
    return jnp.where(v > 0, v, jnp.exp(jnp.where(v > 0, 0.0, v)) - 1.0)


def _dot(a, b):
    return jnp.dot(a, b, preferred_element_type=jnp.float32,
                   precision=lax.Precision.HIGHEST)


def _tc_layer1(x, w1, cnt):
    def body(x_ref, w_ref, ca_ref, cb_ref, t_ref, dinv_ref):
        deg = ca_ref[0, :, 0:1] + cb_ref[0, :, 0:1] - 1.0
        dinv = lax.rsqrt(deg)
        dinv_ref[...] = dinv
        t_ref[...] = _dot(x_ref[...], w_ref[...]) * dinv

    return pl.pallas_call(
        body,
        grid=(N // RB,),
        in_specs=[
            pl.BlockSpec((RB, D), lambda i: (i, 0)),
            pl.BlockSpec((D, D), lambda i: (0, 0)),
            pl.BlockSpec((1, RB, DEGW), lambda i: (0, i, 0)),
            pl.BlockSpec((1, RB, DEGW), lambda i: (1, i, 0)),
        ],
        out_specs=[
            pl.BlockSpec((RB, D), lambda i: (i, 0)),
            pl.BlockSpec((RB, 1), lambda i: (i, 0)),
        ],
        out_shape=[
            jax.ShapeDtypeStruct((N, D), jnp.float32),
            jax.ShapeDtypeStruct((N, 1), jnp.float32),
        ],
    )(x, w1, cnt, cnt)


def _tc_layer_next(part, t_prev, dinv, b_prev, w_next):
    def body(pa_ref, pb_ref, tp_ref, dinv_ref, b_ref, w_ref, t_ref):
        dinv = dinv_ref[...]
        agg = pa_ref[0] + pb_ref[0] - tp_ref[...]
        h = _elu(dinv * agg + b_ref[...])
        t_ref[...] = _dot(h, w_ref[...]) * dinv

    return pl.pallas_call(
        body,
        grid=(N // RB,),
        in_specs=[
            pl.BlockSpec((1, RB, D), lambda i: (0, i, 0)),
            pl.BlockSpec((1, RB, D), lambda i: (1, i, 0)),
            pl.BlockSpec((RB, D), lambda i: (i, 0)),
            pl.BlockSpec((RB, 1), lambda i: (i, 0)),
            pl.BlockSpec((1, D), lambda i: (0, 0)),
            pl.BlockSpec((D, D), lambda i: (0, 0)),
        ],
        out_specs=pl.BlockSpec((RB, D), lambda i: (i, 0)),
        out_shape=jax.ShapeDtypeStruct((N, D), jnp.float32),
    )(part, part, t_prev, dinv, b_prev, w_next)


def _tc_offsets(batch_col):
    def body(b_ref, o_ref):
        hist = jnp.zeros((1, G), jnp.float32)
        for i in range(N // RB):
            vals = b_ref[i * RB:(i + 1) * RB, :]
            eq = (vals == lax.broadcasted_iota(jnp.int32, (RB, G), 1))
            hist = hist + jnp.sum(eq.astype(jnp.float32), axis=0, keepdims=True)
        row = lax.broadcasted_iota(jnp.int32, (G, G), 0)
        col = lax.broadcasted_iota(jnp.int32, (G, G), 1)
        strict_lower = (row < col).astype(jnp.float32)
        starts = _dot(hist, strict_lower)
        o_ref[...] = starts.astype(jnp.int32)

    return pl.pallas_call(
        body,
        in_specs=[pl.BlockSpec((N, 1), lambda: (0, 0))],
        out_specs=pl.BlockSpec((1, G), lambda: (0, 0)),
        out_shape=jax.ShapeDtypeStruct((1, G), jnp.int32),
    )(batch_col)


def _tc_pool_head(part3, t3, dinv, b3, wo, bo, starts):
    def body(starts_ref, pa_ref, pb_ref, tp_ref, dinv_ref, b_ref, wo_ref,
             bo_ref, o_ref, pooled):
        g = pl.program_id(0)

        @pl.when(g < G)
        def _():
            s = starts_ref[g]
            e = jnp.where(g == G - 1, N, starts_ref[jnp.minimum(g + 1, G - 1)])
            c0 = (s // 8) * 8
            nch = (e - c0 + 7) // 8

            def chunk(i, acc):
                r = c0 + 8 * i
                agg = (pa_ref[0, pl.ds(r, 8), :] + pb_ref[0, pl.ds(r, 8), :]
                       - tp_ref[pl.ds(r, 8), :])
                dinv = dinv_ref[pl.ds(r, 8), :]
                h = _elu(dinv * agg + b_ref[...])
                rid = r + lax.broadcasted_iota(jnp.int32, (8, D), 0)
                ok = jnp.logical_and(rid >= s, rid < e)
                return jnp.maximum(acc, jnp.where(ok, h, -1e30))

            acc = lax.fori_loop(0, nch, chunk,
                                jnp.full((8, D), -1e30, jnp.float32))
            pooled[pl.ds(g, 1), :] = jnp.max(acc, axis=0, keepdims=True)

        @pl.when(g == G)
        def _():
            p = pooled[...]
            p = jnp.where(p < -1e29, 0.0, p)
            logits = _dot(p, wo_ref[...]) + bo_ref[...]
            m = jnp.max(logits, axis=1, keepdims=True)
            ex = jnp.exp(logits - m)
            o_ref[...] = ex / jnp.sum(ex, axis=1, keepdims=True)

    grid_spec = pltpu.PrefetchScalarGridSpec(
        num_scalar_prefetch=1,
        grid=(G + 1,),
        in_specs=[
            pl.BlockSpec((1, N, D), lambda g, s_ref: (0, 0, 0)),
            pl.BlockSpec((1, N, D), lambda g, s_ref: (1, 0, 0)),
            pl.BlockSpec((N, D), lambda g, s_ref: (0, 0)),
            pl.BlockSpec((N, 1), lambda g, s_ref: (0, 0)),
            pl.BlockSpec((1, D), lambda g, s_ref: (0, 0)),
            pl.BlockSpec((D, NCLS), lambda g, s_ref: (0, 0)),
            pl.BlockSpec((1, NCLS), lambda g, s_ref: (0, 0)),
        ],
        out_specs=pl.BlockSpec((G, NCLS), lambda g, s_ref: (0, 0)),
        scratch_shapes=[pltpu.VMEM((G, D), jnp.float32)],
    )
    return pl.pallas_call(
        body,
        grid_spec=grid_spec,
        out_shape=jax.ShapeDtypeStruct((G, NCLS), jnp.float32),
    )(starts, part3, part3, t3, dinv, b3, wo, bo)


def kernel(x, edge_index, batch, W1, b1, W2, b2, W3, b3, Wo, bo):
    src_w = jnp.pad(edge_index[0].astype(jnp.int32).reshape(NW, EPW),
                    ((0, 0), (0, PAD))).reshape(NW, CPW, CHUNK)
    dst_w = jnp.pad(edge_index[1].astype(jnp.int32).reshape(NW, EPW),
                    ((0, 0), (0, PAD)), constant_values=N).reshape(NW, CPW, CHUNK)
    batch_col = batch.astype(jnp.int32).reshape(N, 1)
    ones_hbm = jnp.ones((RPT, DEGW), jnp.float32)

    cnt = _sc_degree(ones_hbm, dst_w)
    t1, dinv = _tc_layer1(x, W1, cnt)
    p1 = _sc_aggregate(t1, src_w, dst_w)
    t2 = _tc_layer_next(p1, t1, dinv, b1.reshape(1, D), W2)
    p2 = _sc_aggregate(t2, src_w, dst_w)
    t3 = _tc_layer_next(p2, t2, dinv, b2.reshape(1, D), W3)
    p3 = _sc_aggregate(t3, src_w, dst_w)
    starts = _tc_offsets(batch_col)
    return _tc_pool_head(p3, t3, dinv, b3.reshape(1, D), Wo,
                         bo.reshape(1, NCLS), starts.reshape(G))


# speedup vs baseline: 1.8693x; 1.8693x over previous
import functools

import jax
import jax.numpy as jnp
from jax import lax
from jax.experimental import pallas as pl
from jax.experimental.pallas import tpu as pltpu
from jax.experimental.pallas import tpu_sc as plsc

N = 10000
E = 320000
D = 128
G = 128
NCLS = 10
NC, NS = 2, 16
NW = NC * NS
EPW = E // NW
CHUNK = 128
CPW = 80
PAD = CPW * CHUNK - EPW
ACC_R = N + 8
CHUNK_A = 80
CPW_A = EPW // CHUNK_A
RPT = 624
TAIL = N - NS * RPT
DEGW = 16
RB = 1000

def _mesh():
    return plsc.VectorSubcoreMesh(core_axis_name="c", subcore_axis_name="s",
                                  num_cores=NC, num_subcores=NS)


def _sc_degree(ones_hbm, dst_w):
    @functools.partial(
        pl.kernel,
        out_type=jax.ShapeDtypeStruct((NC, N, DEGW), jnp.float32),
        mesh=_mesh(),
        scratch_types=[
            pltpu.VMEM((CPW, CHUNK), jnp.int32),
            pltpu.VMEM((CHUNK, DEGW), jnp.float32),
            pltpu.SemaphoreType.DMA,
            pltpu.VMEM_SHARED((ACC_R, DEGW), jnp.float32),
        ],
    )
    def k(ones_ref, dst_ref, out_ref, idx_v, ones_v, ssem, acc):
        cid = lax.axis_index("c")
        sid = lax.axis_index("s")
        w = cid * NS + sid
        pltpu.sync_copy(dst_ref.at[w], idx_v)
        pltpu.sync_copy(ones_ref.at[pl.ds(0, CHUNK)], ones_v)
        rs = pl.ds(sid * RPT, RPT)
        ts = pl.ds(NS * RPT, TAIL)
        pltpu.sync_copy(ones_ref.at[pl.ds(0, RPT)], acc.at[rs])

        @pl.when(sid == NS - 1)
        def _():
            pltpu.sync_copy(ones_ref.at[pl.ds(0, TAIL)], acc.at[ts])

        plsc.subcore_barrier()

        @pl.loop(0, CPW)
        def _(j):
            pltpu.async_copy(ones_v, acc.at[idx_v.at[j]], ssem, add=True)

        @pl.loop(0, CPW)
        def _(j):
            pltpu.make_async_copy(ones_v, acc.at[idx_v.at[j]], ssem).wait()

        plsc.subcore_barrier()
        pltpu.sync_copy(acc.at[rs], out_ref.at[cid, rs])

        @pl.when(sid == NS - 1)
        def _():
            pltpu.sync_copy(acc.at[ts], out_ref.at[cid, ts])

    return k(ones_hbm, dst_w)


def _sc_aggregate(t, src_w, dst_w):
    @functools.partial(
        pl.kernel,
        out_type=jax.ShapeDtypeStruct((NC, N, D), jnp.float32),
        mesh=_mesh(),
        scratch_types=[
            pltpu.VMEM((CPW_A, CHUNK_A), jnp.int32),
            pltpu.VMEM((CPW_A, CHUNK_A), jnp.int32),
            pltpu.VMEM((CHUNK_A, D), jnp.float32),
            pltpu.VMEM_SHARED((N, D), jnp.float32),
        ],
    )
    def k(t_ref, src_ref, dst_ref, out_ref, srcv, didx, rowbuf, acc):
        cid = lax.axis_index("c")
        sid = lax.axis_index("s")
        w = cid * NS + sid
        pltpu.sync_copy(src_ref.at[w], srcv)
        pltpu.sync_copy(dst_ref.at[w], didx)
        rs = pl.ds(sid * RPT, RPT)
        ts = pl.ds(NS * RPT, TAIL)
        pltpu.sync_copy(t_ref.at[rs], acc.at[rs])

        @pl.when(sid == NS - 1)
        def _():
            pltpu.sync_copy(t_ref.at[ts], acc.at[ts])

        plsc.subcore_barrier()

        @pl.loop(0, CPW_A)
        def _(j):
            pltpu.sync_copy(t_ref.at[srcv.at[j]], rowbuf)
            pltpu.sync_copy(rowbuf, acc.at[didx.at[j]], add=True)

        plsc.subcore_barrier()
        pltpu.sync_copy(acc.at[rs], out_ref.at[cid, rs])

        @pl.when(sid == NS - 1)
        def _():
            pltpu.sync_copy(acc.at[ts], out_ref.at[cid, ts])

    return k(t, src_w, dst_w)


def _elu(v):
    return jnp.where(v > 0, v, jnp.exp(jnp.where(v > 0, 0.0, v)) - 1.0)


def _dot(a, b):
    return jnp.dot(a, b, preferred_element_type=jnp.float32,
                   precision=lax.Precision.HIGHEST)


def _tc_layer1(x, w1, cnt):
    def body(x_ref, w_ref, ca_ref, cb_ref, t_ref, dinv_ref):
        deg = ca_ref[0, :, 0:1] + cb_ref[0, :, 0:1] - 1.0
        dinv = lax.rsqrt(deg)
        dinv_ref[...] = dinv
        t_ref[...] = _dot(x_ref[...], w_ref[...]) * dinv

    return pl.pallas_call(
        body,
        grid=(N // RB,),
        in_specs=[
            pl.BlockSpec((RB, D), lambda i: (i, 0)),
            pl.BlockSpec((D, D), lambda i: (0, 0)),
            pl.BlockSpec((1, RB, DEGW), lambda i: (0, i, 0)),
            pl.BlockSpec((1, RB, DEGW), lambda i: (1, i, 0)),
        ],
        out_specs=[
            pl.BlockSpec((RB, D), lambda i: (i, 0)),
            pl.BlockSpec((RB, 1), lambda i: (i, 0)),
        ],
        out_shape=[
            jax.ShapeDtypeStruct((N, D), jnp.float32),
            jax.ShapeDtypeStruct((N, 1), jnp.float32),
        ],
    )(x, w1, cnt, cnt)


def _tc_layer_next(part, t_prev, dinv, b_prev, w_next):
    def body(pa_ref, pb_ref, tp_ref, dinv_ref, b_ref, w_ref, t_ref):
        dinv = dinv_ref[...]
        agg = pa_ref[0] + pb_ref[0] - tp_ref[...]
        h = _elu(dinv * agg + b_ref[...])
        t_ref[...] = _dot(h, w_ref[...]) * dinv

    return pl.pallas_call(
        body,
        grid=(N // RB,),
        in_specs=[
            pl.BlockSpec((1, RB, D), lambda i: (0, i, 0)),
            pl.BlockSpec((1, RB, D), lambda i: (1, i, 0)),
            pl.BlockSpec((RB, D), lambda i: (i, 0)),
            pl.BlockSpec((RB, 1), lambda i: (i, 0)),
            pl.BlockSpec((1, D), lambda i: (0, 0)),
            pl.BlockSpec((D, D), lambda i: (0, 0)),
        ],
        out_specs=pl.BlockSpec((RB, D), lambda i: (i, 0)),
        out_shape=jax.ShapeDtypeStruct((N, D), jnp.float32),
    )(part, part, t_prev, dinv, b_prev, w_next)


def _tc_offsets(batch_col):
    def body(b_ref, o_ref):
        hist = jnp.zeros((1, G), jnp.float32)
        for i in range(N // RB):
            vals = b_ref[i * RB:(i + 1) * RB, :]
            eq = (vals == lax.broadcasted_iota(jnp.int32, (RB, G), 1))
            hist = hist + jnp.sum(eq.astype(jnp.float32), axis=0, keepdims=True)
        row = lax.broadcasted_iota(jnp.int32, (G, G), 0)
        col = lax.broadcasted_iota(jnp.int32, (G, G), 1)
        strict_lower = (row < col).astype(jnp.float32)
        starts = _dot(hist, strict_lower)
        o_ref[...] = starts.astype(jnp.int32)

    return pl.pallas_call(
        body,
        in_specs=[pl.BlockSpec((N, 1), lambda: (0, 0))],
        out_specs=pl.BlockSpec((1, G), lambda: (0, 0)),
        out_shape=jax.ShapeDtypeStruct((1, G), jnp.int32),
    )(batch_col)


def _tc_pool_head(part3, t3, dinv, b3, wo, bo, starts):
    def body(starts_ref, pa_ref, pb_ref, tp_ref, dinv_ref, b_ref, wo_ref,
             bo_ref, o_ref, pooled):
        g = pl.program_id(0)

        @pl.when(g < G)
        def _():
            s = starts_ref[g]
            e = jnp.where(g == G - 1, N, starts_ref[jnp.minimum(g + 1, G - 1)])
            c0 = (s // 8) * 8
            nch = (e - c0 + 7) // 8

            def chunk(i, acc):
                r = c0 + 8 * i
                agg = (pa_ref[0, pl.ds(r, 8), :] + pb_ref[0, pl.ds(r, 8), :]
                       - tp_ref[pl.ds(r, 8), :])
                dinv = dinv_ref[pl.ds(r, 8), :]
                h = _elu(dinv * agg + b_ref[...])
                rid = r + lax.broadcasted_iota(jnp.int32, (8, D), 0)
                ok = jnp.logical_and(rid >= s, rid < e)
                return jnp.maximum(acc, jnp.where(ok, h, -1e30))

            acc = lax.fori_loop(0, nch, chunk,
                                jnp.full((8, D), -1e30, jnp.float32))
            pooled[pl.ds(g, 1), :] = jnp.max(acc, axis=0, keepdims=True)

        @pl.when(g == G)
        def _():
            p = pooled[...]
            p = jnp.where(p < -1e29, 0.0, p)
            logits = _dot(p, wo_ref[...]) + bo_ref[...]
            m = jnp.max(logits, axis=1, keepdims=True)
            ex = jnp.exp(logits - m)
            o_ref[...] = ex / jnp.sum(ex, axis=1, keepdims=True)

    grid_spec = pltpu.PrefetchScalarGridSpec(
        num_scalar_prefetch=1,
        grid=(G + 1,),
        in_specs=[
            pl.BlockSpec((1, N, D), lambda g, s_ref: (0, 0, 0)),
            pl.BlockSpec((1, N, D), lambda g, s_ref: (1, 0, 0)),
            pl.BlockSpec((N, D), lambda g, s_ref: (0, 0)),
            pl.BlockSpec((N, 1), lambda g, s_ref: (0, 0)),
            pl.BlockSpec((1, D), lambda g, s_ref: (0, 0)),
            pl.BlockSpec((D, NCLS), lambda g, s_ref: (0, 0)),
            pl.BlockSpec((1, NCLS), lambda g, s_ref: (0, 0)),
        ],
        out_specs=pl.BlockSpec((G, NCLS), lambda g, s_ref: (0, 0)),
        scratch_shapes=[pltpu.VMEM((G, D), jnp.float32)],
    )
    return pl.pallas_call(
        body,
        grid_spec=grid_spec,
        out_shape=jax.ShapeDtypeStruct((G, NCLS), jnp.float32),
    )(starts, part3, part3, t3, dinv, b3, wo, bo)


def kernel(x, edge_index, batch, W1, b1, W2, b2, W3, b3, Wo, bo):
    src32 = edge_index[0].astype(jnp.int32).reshape(NW, EPW)
    dst32 = edge_index[1].astype(jnp.int32).reshape(NW, EPW)
    src_w = src32.reshape(NW, CPW_A, CHUNK_A)
    dst_w = dst32.reshape(NW, CPW_A, CHUNK_A)
    dst_deg = jnp.pad(dst32, ((0, 0), (0, PAD)),
                      constant_values=N).reshape(NW, CPW, CHUNK)
    batch_col = batch.astype(jnp.int32).reshape(N, 1)
    ones_hbm = jnp.ones((RPT, DEGW), jnp.float32)

    cnt = _sc_degree(ones_hbm, dst_deg)
    t1, dinv = _tc_layer1(x, W1, cnt)
    p1 = _sc_aggregate(t1, src_w, dst_w)
    t2 = _tc_layer_next(p1, t1, dinv, b1.reshape(1, D), W2)
    p2 = _sc_aggregate(t2, src_w, dst_w)
    t3 = _tc_layer_next(p2, t2, dinv, b2.reshape(1, D), W3)
    p3 = _sc_aggregate(t3, src_w, dst_w)
    starts = _tc_offsets(batch_col)
    return _tc_pool_head(p3, t3, dinv, b3.reshape(1, D), Wo,
                         bo.reshape(1, NCLS), starts.reshape(G))


# speedup vs baseline: 2.1172x; 1.1326x over previous
import functools

import jax
import jax.numpy as jnp
from jax import lax
from jax.experimental import pallas as pl
from jax.experimental.pallas import tpu as pltpu
from jax.experimental.pallas import tpu_sc as plsc

N = 10000
E = 320000
D = 128
G = 128
NCLS = 10
NC, NS = 2, 16
NW = NC * NS
EPW = E // NW
CHUNK = 128
CPW = 80
PAD = CPW * CHUNK - EPW
ACC_R = N + 8
CHUNK_A = 80
CPW_A = EPW // CHUNK_A
RPT = 624
TAIL = N - NS * RPT
DEGW = 16
RB = 1000
GPT = G // NW
CB = 128
RB_H = 1024
N_H = 10240

def _mesh():
    return plsc.VectorSubcoreMesh(core_axis_name="c", subcore_axis_name="s",
                                  num_cores=NC, num_subcores=NS)


def _sc_degree(ones_hbm, dst_w):
    @functools.partial(
        pl.kernel,
        out_type=jax.ShapeDtypeStruct((NC, N, DEGW), jnp.float32),
        mesh=_mesh(),
        scratch_types=[
            pltpu.VMEM((CPW, CHUNK), jnp.int32),
            pltpu.VMEM((CHUNK, DEGW), jnp.float32),
            pltpu.SemaphoreType.DMA,
            pltpu.VMEM_SHARED((ACC_R, DEGW), jnp.float32),
        ],
    )
    def k(ones_ref, dst_ref, out_ref, idx_v, ones_v, ssem, acc):
        cid = lax.axis_index("c")
        sid = lax.axis_index("s")
        w = cid * NS + sid
        pltpu.sync_copy(dst_ref.at[w], idx_v)
        pltpu.sync_copy(ones_ref.at[pl.ds(0, CHUNK)], ones_v)
        rs = pl.ds(sid * RPT, RPT)
        ts = pl.ds(NS * RPT, TAIL)
        pltpu.sync_copy(ones_ref.at[pl.ds(0, RPT)], acc.at[rs])

        @pl.when(sid == NS - 1)
        def _():
            pltpu.sync_copy(ones_ref.at[pl.ds(0, TAIL)], acc.at[ts])

        plsc.subcore_barrier()

        @pl.loop(0, CPW)
        def _(j):
            pltpu.async_copy(ones_v, acc.at[idx_v.at[j]], ssem, add=True)

        @pl.loop(0, CPW)
        def _(j):
            pltpu.make_async_copy(ones_v, acc.at[idx_v.at[j]], ssem).wait()

        plsc.subcore_barrier()
        pltpu.sync_copy(acc.at[rs], out_ref.at[cid, rs])

        @pl.when(sid == NS - 1)
        def _():
            pltpu.sync_copy(acc.at[ts], out_ref.at[cid, ts])

    return k(ones_hbm, dst_w)


def _sc_aggregate(t, src_w, dst_w):
    @functools.partial(
        pl.kernel,
        out_type=jax.ShapeDtypeStruct((NC, N, D), jnp.float32),
        mesh=_mesh(),
        scratch_types=[
            pltpu.VMEM((CPW_A, CHUNK_A), jnp.int32),
            pltpu.VMEM((CPW_A, CHUNK_A), jnp.int32),
            pltpu.VMEM((CHUNK_A, D), jnp.float32),
            pltpu.VMEM_SHARED((N, D), jnp.float32),
        ],
    )
    def k(t_ref, src_ref, dst_ref, out_ref, srcv, didx, rowbuf, acc):
        cid = lax.axis_index("c")
        sid = lax.axis_index("s")
        w = cid * NS + sid
        pltpu.sync_copy(src_ref.at[w], srcv)
        pltpu.sync_copy(dst_ref.at[w], didx)
        rs = pl.ds(sid * RPT, RPT)
        ts = pl.ds(NS * RPT, TAIL)
        pltpu.sync_copy(t_ref.at[rs], acc.at[rs])

        @pl.when(sid == NS - 1)
        def _():
            pltpu.sync_copy(t_ref.at[ts], acc.at[ts])

        plsc.subcore_barrier()

        @pl.loop(0, CPW_A)
        def _(j):
            pltpu.sync_copy(t_ref.at[srcv.at[j]], rowbuf)
            pltpu.sync_copy(rowbuf, acc.at[didx.at[j]], add=True)

        plsc.subcore_barrier()
        pltpu.sync_copy(acc.at[rs], out_ref.at[cid, rs])

        @pl.when(sid == NS - 1)
        def _():
            pltpu.sync_copy(acc.at[ts], out_ref.at[cid, ts])

    return k(t, src_w, dst_w)


def _elu(v):
    return jnp.where(v > 0, v, jnp.exp(jnp.where(v > 0, 0.0, v)) - 1.0)


def _dot(a, b):
    return jnp.dot(a, b, preferred_element_type=jnp.float32,
                   precision=lax.Precision.HIGHEST)


def _tc_layer1(x, w1, cnt):
    def body(x_ref, w_ref, ca_ref, cb_ref, t_ref, dinv_ref):
        deg = ca_ref[0, :, 0:1] + cb_ref[0, :, 0:1] - 1.0
        dinv = lax.rsqrt(deg)
        dinv_ref[...] = dinv
        t_ref[...] = _dot(x_ref[...], w_ref[...]) * dinv

    return pl.pallas_call(
        body,
        grid=(N // RB,),
        in_specs=[
            pl.BlockSpec((RB, D), lambda i: (i, 0)),
            pl.BlockSpec((D, D), lambda i: (0, 0)),
            pl.BlockSpec((1, RB, DEGW), lambda i: (0, i, 0)),
            pl.BlockSpec((1, RB, DEGW), lambda i: (1, i, 0)),
        ],
        out_specs=[
            pl.BlockSpec((RB, D), lambda i: (i, 0)),
            pl.BlockSpec((RB, 1), lambda i: (i, 0)),
        ],
        out_shape=[
            jax.ShapeDtypeStruct((N, D), jnp.float32),
            jax.ShapeDtypeStruct((N, 1), jnp.float32),
        ],
    )(x, w1, cnt, cnt)


def _tc_layer_next(part, t_prev, dinv, b_prev, w_next):
    def body(pa_ref, pb_ref, tp_ref, dinv_ref, b_ref, w_ref, t_ref):
        dinv = dinv_ref[...]
        agg = pa_ref[0] + pb_ref[0] - tp_ref[...]
        h = _elu(dinv * agg + b_ref[...])
        t_ref[...] = _dot(h, w_ref[...]) * dinv

    return pl.pallas_call(
        body,
        grid=(N // RB,),
        in_specs=[
            pl.BlockSpec((1, RB, D), lambda i: (0, i, 0)),
            pl.BlockSpec((1, RB, D), lambda i: (1, i, 0)),
            pl.BlockSpec((RB, D), lambda i: (i, 0)),
            pl.BlockSpec((RB, 1), lambda i: (i, 0)),
            pl.BlockSpec((1, D), lambda i: (0, 0)),
            pl.BlockSpec((D, D), lambda i: (0, 0)),
        ],
        out_specs=pl.BlockSpec((RB, D), lambda i: (i, 0)),
        out_shape=jax.ShapeDtypeStruct((N, D), jnp.float32),
    )(part, part, t_prev, dinv, b_prev, w_next)


def _tc_offsets(batch_col):
    def body(b_ref, o_ref):
        hist = jnp.zeros((1, G), jnp.float32)
        for i in range(N // RB):
            vals = b_ref[i * RB:(i + 1) * RB, :]
            eq = (vals == lax.broadcasted_iota(jnp.int32, (RB, G), 1))
            hist = hist + jnp.sum(eq.astype(jnp.float32), axis=0, keepdims=True)
        row = lax.broadcasted_iota(jnp.int32, (G, G), 0)
        col = lax.broadcasted_iota(jnp.int32, (G, G), 1)
        strict_lower = (row < col).astype(jnp.float32)
        starts = _dot(hist, strict_lower)
        o_ref[...] = starts.astype(jnp.int32)

    return pl.pallas_call(
        body,
        in_specs=[pl.BlockSpec((N, 1), lambda: (0, 0))],
        out_specs=pl.BlockSpec((1, G), lambda: (0, 0)),
        out_shape=jax.ShapeDtypeStruct((1, G), jnp.int32),
    )(batch_col)


def _tc_h3(part3, t3, dinv, b3):
    def body(pa_ref, pb_ref, tp_ref, dinv_ref, b_ref, h_ref):
        agg = pa_ref[0] + pb_ref[0] - tp_ref[...]
        h = _elu(dinv_ref[...] * agg + b_ref[...])
        rid = (pl.program_id(0) * RB_H
               + lax.broadcasted_iota(jnp.int32, (RB_H, D), 0))
        h_ref[...] = jnp.where(rid < N, h, -1e30)

    return pl.pallas_call(
        body,
        grid=(N_H // RB_H,),
        in_specs=[
            pl.BlockSpec((1, RB_H, D), lambda i: (0, i, 0)),
            pl.BlockSpec((1, RB_H, D), lambda i: (1, i, 0)),
            pl.BlockSpec((RB_H, D), lambda i: (i, 0)),
            pl.BlockSpec((RB_H, 1), lambda i: (i, 0)),
            pl.BlockSpec((1, D), lambda i: (0, 0)),
        ],
        out_specs=pl.BlockSpec((RB_H, D), lambda i: (i, 0)),
        out_shape=jax.ShapeDtypeStruct((N_H, D), jnp.float32),
    )(part3, part3, t3, dinv, b3)


def _sc_pool(h3, starts_ext):
    @functools.partial(
        pl.kernel,
        out_type=jax.ShapeDtypeStruct((NW, 8, D), jnp.float32),
        mesh=_mesh(),
        scratch_types=[
            pltpu.VMEM((G + 32,), jnp.int32),
            pltpu.VMEM((CB, D), jnp.float32),
            pltpu.VMEM((8, D), jnp.float32),
        ],
    )
    def k(h_ref, st_ref, out_ref, smem, buf, vout):
        cid = lax.axis_index("c")
        sid = lax.axis_index("s")
        w = cid * NS + sid
        pltpu.sync_copy(st_ref, smem)
        neg = jnp.full((16,), -1e30, jnp.float32)
        for gi in range(8):
            for f in range(8):
                vout[gi, pl.ds(16 * f, 16)] = neg
        for gi in range(GPT):
            g = w * GPT + gi
            sv = smem[pl.ds(g, 16)]
            s_g = sv[0]
            e_g = sv[1]
            c0 = (s_g // 8) * 8
            nch = (e_g - c0 + CB - 1) // CB

            def chunk_body(c, accs, c0=c0, s_g=s_g, e_g=e_g):
                base = pl.multiple_of(c0 + c * CB, 8)
                pltpu.sync_copy(h_ref.at[pl.ds(base, CB)], buf)
                rlo = jnp.maximum(base, s_g)
                rhi = jnp.minimum(base + CB, e_g)

                def row_body(r, a, base=base):
                    loc = r - base
                    return tuple(
                        jnp.maximum(a[f], buf[loc, pl.ds(16 * f, 16)])
                        for f in range(8))

                return lax.fori_loop(rlo, rhi, row_body, accs)

            accs = lax.fori_loop(0, nch, chunk_body, (neg,) * 8)
            for f in range(8):
                vout[gi, pl.ds(16 * f, 16)] = accs[f]
        pltpu.sync_copy(vout, out_ref.at[w])

    return k(h3, starts_ext)


def _tc_head(pooled_flat, wo, bo):
    def body(p_ref, wo_ref, bo_ref, o_ref):
        row = lax.broadcasted_iota(jnp.int32, (G, NW * 8), 0)
        col = lax.broadcasted_iota(jnp.int32, (G, NW * 8), 1)
        sel = (col == 8 * (row // GPT) + row % GPT).astype(jnp.float32)
        pooled = _dot(sel, p_ref[...])
        pooled = jnp.where(pooled < -1e29, 0.0, pooled)
        logits = _dot(pooled, wo_ref[...]) + bo_ref[...]
        m = jnp.max(logits, axis=1, keepdims=True)
        ex = jnp.exp(logits - m)
        o_ref[...] = ex / jnp.sum(ex, axis=1, keepdims=True)

    return pl.pallas_call(
        body,
        in_specs=[
            pl.BlockSpec((NW * 8, D), lambda: (0, 0)),
            pl.BlockSpec((D, NCLS), lambda: (0, 0)),
            pl.BlockSpec((1, NCLS), lambda: (0, 0)),
        ],
        out_specs=pl.BlockSpec((G, NCLS), lambda: (0, 0)),
        out_shape=jax.ShapeDtypeStruct((G, NCLS), jnp.float32),
    )(pooled_flat, wo, bo)


def kernel(x, edge_index, batch, W1, b1, W2, b2, W3, b3, Wo, bo):
    src32 = edge_index[0].astype(jnp.int32).reshape(NW, EPW)
    dst32 = edge_index[1].astype(jnp.int32).reshape(NW, EPW)
    src_w = src32.reshape(NW, CPW_A, CHUNK_A)
    dst_w = dst32.reshape(NW, CPW_A, CHUNK_A)
    dst_deg = jnp.pad(dst32, ((0, 0), (0, PAD)),
                      constant_values=N).reshape(NW, CPW, CHUNK)
    batch_col = batch.astype(jnp.int32).reshape(N, 1)
    ones_hbm = jnp.ones((RPT, DEGW), jnp.float32)

    cnt = _sc_degree(ones_hbm, dst_deg)
    t1, dinv = _tc_layer1(x, W1, cnt)
    p1 = _sc_aggregate(t1, src_w, dst_w)
    t2 = _tc_layer_next(p1, t1, dinv, b1.reshape(1, D), W2)
    p2 = _sc_aggregate(t2, src_w, dst_w)
    t3 = _tc_layer_next(p2, t2, dinv, b2.reshape(1, D), W3)
    p3 = _sc_aggregate(t3, src_w, dst_w)
    starts = _tc_offsets(batch_col)
    h3 = _tc_h3(p3, t3, dinv, b3.reshape(1, D))
    starts_ext = jnp.concatenate(
        [starts.reshape(G), jnp.full((32,), N, jnp.int32)])
    pooled_b = _sc_pool(h3, starts_ext)
    return _tc_head(pooled_b.reshape(NW * 8, D), Wo, bo.reshape(1, NCLS))


# speedup vs baseline: 2.4164x; 1.1414x over previous
import functools

import jax
import jax.numpy as jnp
from jax import lax
from jax.experimental import pallas as pl
from jax.experimental.pallas import tpu as pltpu
from jax.experimental.pallas import tpu_sc as plsc

N = 10000
E = 320000
D = 128
G = 128
NCLS = 10
NC, NS = 2, 16
NW = NC * NS
EPW = E // NW
CHUNK = 128
CPW = 80
PAD = CPW * CHUNK - EPW
ACC_R = N + 8
CHUNK_A = 125
CPW_A = EPW // CHUNK_A
RPT = 624
TAIL = N - NS * RPT
DEGW = 16
RB = 1000
GPT = G // NW
CB = 128
RB_H = 1024
N_H = 10240

def _mesh():
    return plsc.VectorSubcoreMesh(core_axis_name="c", subcore_axis_name="s",
                                  num_cores=NC, num_subcores=NS)


def _sc_degree(ones_hbm, dst_w):
    @functools.partial(
        pl.kernel,
        out_type=jax.ShapeDtypeStruct((NC, N, DEGW), jnp.float32),
        mesh=_mesh(),
        scratch_types=[
            pltpu.VMEM((CPW, CHUNK), jnp.int32),
            pltpu.VMEM((CHUNK, DEGW), jnp.float32),
            pltpu.SemaphoreType.DMA,
            pltpu.VMEM_SHARED((ACC_R, DEGW), jnp.float32),
        ],
    )
    def k(ones_ref, dst_ref, out_ref, idx_v, ones_v, ssem, acc):
        cid = lax.axis_index("c")
        sid = lax.axis_index("s")
        w = cid * NS + sid
        pltpu.sync_copy(dst_ref.at[w], idx_v)
        pltpu.sync_copy(ones_ref.at[pl.ds(0, CHUNK)], ones_v)
        rs = pl.ds(sid * RPT, RPT)
        ts = pl.ds(NS * RPT, TAIL)
        pltpu.sync_copy(ones_ref.at[pl.ds(0, RPT)], acc.at[rs])

        @pl.when(sid == NS - 1)
        def _():
            pltpu.sync_copy(ones_ref.at[pl.ds(0, TAIL)], acc.at[ts])

        plsc.subcore_barrier()

        @pl.loop(0, CPW)
        def _(j):
            pltpu.async_copy(ones_v, acc.at[idx_v.at[j]], ssem, add=True)

        @pl.loop(0, CPW)
        def _(j):
            pltpu.make_async_copy(ones_v, acc.at[idx_v.at[j]], ssem).wait()

        plsc.subcore_barrier()
        pltpu.sync_copy(acc.at[rs], out_ref.at[cid, rs])

        @pl.when(sid == NS - 1)
        def _():
            pltpu.sync_copy(acc.at[ts], out_ref.at[cid, ts])

    return k(ones_hbm, dst_w)


def _sc_aggregate(t, src_w, dst_w):
    @functools.partial(
        pl.kernel,
        out_type=jax.ShapeDtypeStruct((NC, N, D), jnp.float32),
        mesh=_mesh(),
        scratch_types=[
            pltpu.VMEM((CPW_A, CHUNK_A), jnp.int32),
            pltpu.VMEM((CPW_A, CHUNK_A), jnp.int32),
            pltpu.VMEM((CHUNK_A, D), jnp.float32),
            pltpu.VMEM_SHARED((N, D), jnp.float32),
        ],
    )
    def k(t_ref, src_ref, dst_ref, out_ref, srcv, didx, rowbuf, acc):
        cid = lax.axis_index("c")
        sid = lax.axis_index("s")
        w = cid * NS + sid
        pltpu.sync_copy(src_ref.at[w], srcv)
        pltpu.sync_copy(dst_ref.at[w], didx)
        rs = pl.ds(sid * RPT, RPT)
        ts = pl.ds(NS * RPT, TAIL)
        pltpu.sync_copy(t_ref.at[rs], acc.at[rs])

        @pl.when(sid == NS - 1)
        def _():
            pltpu.sync_copy(t_ref.at[ts], acc.at[ts])

        plsc.subcore_barrier()

        @pl.loop(0, CPW_A)
        def _(j):
            pltpu.sync_copy(t_ref.at[srcv.at[j]], rowbuf)
            pltpu.sync_copy(rowbuf, acc.at[didx.at[j]], add=True)

        plsc.subcore_barrier()
        pltpu.sync_copy(acc.at[rs], out_ref.at[cid, rs])

        @pl.when(sid == NS - 1)
        def _():
            pltpu.sync_copy(acc.at[ts], out_ref.at[cid, ts])

    return k(t, src_w, dst_w)


def _elu(v):
    return jnp.where(v > 0, v, jnp.exp(jnp.where(v > 0, 0.0, v)) - 1.0)


def _dot(a, b):
    return jnp.dot(a, b, preferred_element_type=jnp.float32,
                   precision=lax.Precision.HIGHEST)


def _tc_layer1(x, w1, cnt):
    def body(x_ref, w_ref, ca_ref, cb_ref, t_ref, dinv_ref):
        deg = ca_ref[0, :, 0:1] + cb_ref[0, :, 0:1] - 1.0
        dinv = lax.rsqrt(deg)
        dinv_ref[...] = dinv
        t_ref[...] = _dot(x_ref[...], w_ref[...]) * dinv

    return pl.pallas_call(
        body,
        grid=(N // RB,),
        in_specs=[
            pl.BlockSpec((RB, D), lambda i: (i, 0)),
            pl.BlockSpec((D, D), lambda i: (0, 0)),
            pl.BlockSpec((1, RB, DEGW), lambda i: (0, i, 0)),
            pl.BlockSpec((1, RB, DEGW), lambda i: (1, i, 0)),
        ],
        out_specs=[
            pl.BlockSpec((RB, D), lambda i: (i, 0)),
            pl.BlockSpec((RB, 1), lambda i: (i, 0)),
        ],
        out_shape=[
            jax.ShapeDtypeStruct((N, D), jnp.float32),
            jax.ShapeDtypeStruct((N, 1), jnp.float32),
        ],
    )(x, w1, cnt, cnt)


def _tc_layer_next(part, t_prev, dinv, b_prev, w_next):
    def body(pa_ref, pb_ref, tp_ref, dinv_ref, b_ref, w_ref, t_ref):
        dinv = dinv_ref[...]
        agg = pa_ref[0] + pb_ref[0] - tp_ref[...]
        h = _elu(dinv * agg + b_ref[...])
        t_ref[...] = _dot(h, w_ref[...]) * dinv

    return pl.pallas_call(
        body,
        grid=(N // RB,),
        in_specs=[
            pl.BlockSpec((1, RB, D), lambda i: (0, i, 0)),
            pl.BlockSpec((1, RB, D), lambda i: (1, i, 0)),
            pl.BlockSpec((RB, D), lambda i: (i, 0)),
            pl.BlockSpec((RB, 1), lambda i: (i, 0)),
            pl.BlockSpec((1, D), lambda i: (0, 0)),
            pl.BlockSpec((D, D), lambda i: (0, 0)),
        ],
        out_specs=pl.BlockSpec((RB, D), lambda i: (i, 0)),
        out_shape=jax.ShapeDtypeStruct((N, D), jnp.float32),
    )(part, part, t_prev, dinv, b_prev, w_next)


def _tc_offsets(batch_col):
    def body(b_ref, o_ref):
        hist = jnp.zeros((1, G), jnp.float32)
        for i in range(N // RB):
            vals = b_ref[i * RB:(i + 1) * RB, :]
            eq = (vals == lax.broadcasted_iota(jnp.int32, (RB, G), 1))
            hist = hist + jnp.sum(eq.astype(jnp.float32), axis=0, keepdims=True)
        row = lax.broadcasted_iota(jnp.int32, (G, G), 0)
        col = lax.broadcasted_iota(jnp.int32, (G, G), 1)
        strict_lower = (row < col).astype(jnp.float32)
        starts = _dot(hist, strict_lower)
        o_ref[...] = starts.astype(jnp.int32)

    return pl.pallas_call(
        body,
        in_specs=[pl.BlockSpec((N, 1), lambda: (0, 0))],
        out_specs=pl.BlockSpec((1, G), lambda: (0, 0)),
        out_shape=jax.ShapeDtypeStruct((1, G), jnp.int32),
    )(batch_col)


def _tc_h3(part3, t3, dinv, b3):
    def body(pa_ref, pb_ref, tp_ref, dinv_ref, b_ref, h_ref):
        agg = pa_ref[0] + pb_ref[0] - tp_ref[...]
        h = _elu(dinv_ref[...] * agg + b_ref[...])
        rid = (pl.program_id(0) * RB_H
               + lax.broadcasted_iota(jnp.int32, (RB_H, D), 0))
        h_ref[...] = jnp.where(rid < N, h, -1e30)

    return pl.pallas_call(
        body,
        grid=(N_H // RB_H,),
        in_specs=[
            pl.BlockSpec((1, RB_H, D), lambda i: (0, i, 0)),
            pl.BlockSpec((1, RB_H, D), lambda i: (1, i, 0)),
            pl.BlockSpec((RB_H, D), lambda i: (i, 0)),
            pl.BlockSpec((RB_H, 1), lambda i: (i, 0)),
            pl.BlockSpec((1, D), lambda i: (0, 0)),
        ],
        out_specs=pl.BlockSpec((RB_H, D), lambda i: (i, 0)),
        out_shape=jax.ShapeDtypeStruct((N_H, D), jnp.float32),
    )(part3, part3, t3, dinv, b3)


def _sc_pool(h3, starts_ext):
    @functools.partial(
        pl.kernel,
        out_type=jax.ShapeDtypeStruct((NW, 8, D), jnp.float32),
        mesh=_mesh(),
        scratch_types=[
            pltpu.VMEM((G + 32,), jnp.int32),
            pltpu.VMEM((CB, D), jnp.float32),
            pltpu.VMEM((8, D), jnp.float32),
        ],
    )
    def k(h_ref, st_ref, out_ref, smem, buf, vout):
        cid = lax.axis_index("c")
        sid = lax.axis_index("s")
        w = cid * NS + sid
        pltpu.sync_copy(st_ref, smem)
        neg = jnp.full((16,), -1e30, jnp.float32)
        for gi in range(8):
            for f in range(8):
                vout[gi, pl.ds(16 * f, 16)] = neg
        for gi in range(GPT):
            g = w * GPT + gi
            sv = smem[pl.ds(g, 16)]
            s_g = sv[0]
            e_g = sv[1]
            c0 = (s_g // 8) * 8
            nch = (e_g - c0 + CB - 1) // CB

            def chunk_body(c, accs, c0=c0, s_g=s_g, e_g=e_g):
                base = pl.multiple_of(c0 + c * CB, 8)
                pltpu.sync_copy(h_ref.at[pl.ds(base, CB)], buf)
                rlo = jnp.maximum(base, s_g)
                rhi = jnp.minimum(base + CB, e_g)

                def row_body(r, a, base=base):
                    loc = r - base
                    return tuple(
                        jnp.maximum(a[f], buf[loc, pl.ds(16 * f, 16)])
                        for f in range(8))

                return lax.fori_loop(rlo, rhi, row_body, accs)

            accs = lax.fori_loop(0, nch, chunk_body, (neg,) * 8)
            for f in range(8):
                vout[gi, pl.ds(16 * f, 16)] = accs[f]
        pltpu.sync_copy(vout, out_ref.at[w])

    return k(h3, starts_ext)


def _tc_head(pooled_flat, wo, bo):
    def body(p_ref, wo_ref, bo_ref, o_ref):
        row = lax.broadcasted_iota(jnp.int32, (G, NW * 8), 0)
        col = lax.broadcasted_iota(jnp.int32, (G, NW * 8), 1)
        sel = (col == 8 * (row // GPT) + row % GPT).astype(jnp.float32)
        pooled = _dot(sel, p_ref[...])
        pooled = jnp.where(pooled < -1e29, 0.0, pooled)
        logits = _dot(pooled, wo_ref[...]) + bo_ref[...]
        m = jnp.max(logits, axis=1, keepdims=True)
        ex = jnp.exp(logits - m)
        o_ref[...] = ex / jnp.sum(ex, axis=1, keepdims=True)

    return pl.pallas_call(
        body,
        in_specs=[
            pl.BlockSpec((NW * 8, D), lambda: (0, 0)),
            pl.BlockSpec((D, NCLS), lambda: (0, 0)),
            pl.BlockSpec((1, NCLS), lambda: (0, 0)),
        ],
        out_specs=pl.BlockSpec((G, NCLS), lambda: (0, 0)),
        out_shape=jax.ShapeDtypeStruct((G, NCLS), jnp.float32),
    )(pooled_flat, wo, bo)


def kernel(x, edge_index, batch, W1, b1, W2, b2, W3, b3, Wo, bo):
    src32 = edge_index[0].astype(jnp.int32).reshape(NW, EPW)
    dst32 = edge_index[1].astype(jnp.int32).reshape(NW, EPW)
    src_w = src32.reshape(NW, CPW_A, CHUNK_A)
    dst_w = dst32.reshape(NW, CPW_A, CHUNK_A)
    dst_deg = jnp.pad(dst32, ((0, 0), (0, PAD)),
                      constant_values=N).reshape(NW, CPW, CHUNK)
    batch_col = batch.astype(jnp.int32).reshape(N, 1)
    ones_hbm = jnp.ones((RPT, DEGW), jnp.float32)

    cnt = _sc_degree(ones_hbm, dst_deg)
    t1, dinv = _tc_layer1(x, W1, cnt)
    p1 = _sc_aggregate(t1, src_w, dst_w)
    t2 = _tc_layer_next(p1, t1, dinv, b1.reshape(1, D), W2)
    p2 = _sc_aggregate(t2, src_w, dst_w)
    t3 = _tc_layer_next(p2, t2, dinv, b2.reshape(1, D), W3)
    p3 = _sc_aggregate(t3, src_w, dst_w)
    starts = _tc_offsets(batch_col)
    h3 = _tc_h3(p3, t3, dinv, b3.reshape(1, D))
    starts_ext = jnp.concatenate(
        [starts.reshape(G), jnp.full((32,), N, jnp.int32)])
    pooled_b = _sc_pool(h3, starts_ext)
    return _tc_head(pooled_b.reshape(NW * 8, D), Wo, bo.reshape(1, NCLS))


# speedup vs baseline: 3.0430x; 1.2593x over previous
import functools

import jax
import jax.numpy as jnp
from jax import lax
from jax.experimental import pallas as pl
from jax.experimental.pallas import tpu as pltpu
from jax.experimental.pallas import tpu_sc as plsc

N = 10000
E = 320000
D = 128
G = 128
NCLS = 10
NC, NS = 2, 16
NW = NC * NS
EPW = E // NW
CHUNK = 128
CPW = 80
PAD = CPW * CHUNK - EPW
ACC_R = N + 8
CHUNK_A = 125
CPW_A = EPW // CHUNK_A
RPT = 624
TAIL = N - NS * RPT
DEGW = 16
RB = 1000
GPT = G // NW
CB = 128
RB_H = 1024
N_H = 10240

def _mesh():
    return plsc.VectorSubcoreMesh(core_axis_name="c", subcore_axis_name="s",
                                  num_cores=NC, num_subcores=NS)


def _sc_degree(ones_hbm, dst_w):
    @functools.partial(
        pl.kernel,
        out_type=jax.ShapeDtypeStruct((NC, N, DEGW), jnp.float32),
        mesh=_mesh(),
        scratch_types=[
            pltpu.VMEM((CPW, CHUNK), jnp.int32),
            pltpu.VMEM((CHUNK, DEGW), jnp.float32),
            pltpu.SemaphoreType.DMA,
            pltpu.VMEM_SHARED((ACC_R, DEGW), jnp.float32),
        ],
    )
    def k(ones_ref, dst_ref, out_ref, idx_v, ones_v, ssem, acc):
        cid = lax.axis_index("c")
        sid = lax.axis_index("s")
        w = cid * NS + sid
        pltpu.sync_copy(dst_ref.at[w], idx_v)
        pltpu.sync_copy(ones_ref.at[pl.ds(0, CHUNK)], ones_v)
        rs = pl.ds(sid * RPT, RPT)
        ts = pl.ds(NS * RPT, TAIL)
        pltpu.sync_copy(ones_ref.at[pl.ds(0, RPT)], acc.at[rs])

        @pl.when(sid == NS - 1)
        def _():
            pltpu.sync_copy(ones_ref.at[pl.ds(0, TAIL)], acc.at[ts])

        plsc.subcore_barrier()

        @pl.loop(0, CPW)
        def _(j):
            pltpu.async_copy(ones_v, acc.at[idx_v.at[j]], ssem, add=True)

        @pl.loop(0, CPW)
        def _(j):
            pltpu.make_async_copy(ones_v, acc.at[idx_v.at[j]], ssem).wait()

        plsc.subcore_barrier()
        pltpu.sync_copy(acc.at[rs], out_ref.at[cid, rs])

        @pl.when(sid == NS - 1)
        def _():
            pltpu.sync_copy(acc.at[ts], out_ref.at[cid, ts])

    return k(ones_hbm, dst_w)


def _sc_aggregate(t, src_w, dst_w):
    @functools.partial(
        pl.kernel,
        out_type=jax.ShapeDtypeStruct((NC, N, D), jnp.float32),
        mesh=_mesh(),
        scratch_types=[
            pltpu.VMEM((2, 8, CHUNK_A), jnp.int32),
            pltpu.VMEM((CPW_A, CHUNK_A), jnp.int32),
            pltpu.VMEM((CHUNK_A, D), jnp.float32),
            pltpu.VMEM((CHUNK_A, D), jnp.float32),
            pltpu.SemaphoreType.DMA((2,)),
            pltpu.SemaphoreType.DMA((2,)),
            pltpu.VMEM_SHARED((N, D), jnp.float32),
        ],
    )
    def k(t_ref, src_ref, dst_ref, out_ref, srcv, didx, buf0, buf1,
          gsem, isem, acc):
        bufs = (buf0, buf1)
        cid = lax.axis_index("c")
        sid = lax.axis_index("s")
        w = cid * NS + sid
        pltpu.sync_copy(dst_ref.at[w], didx)
        rs = pl.ds(sid * RPT, RPT)
        ts = pl.ds(NS * RPT, TAIL)
        pltpu.sync_copy(t_ref.at[rs], acc.at[rs])

        @pl.when(sid == NS - 1)
        def _():
            pltpu.sync_copy(t_ref.at[ts], acc.at[ts])

        plsc.subcore_barrier()

        pltpu.sync_copy(src_ref.at[w, pl.ds(0, 8)], srcv.at[0])
        pltpu.async_copy(src_ref.at[w, pl.ds(8, 8)], srcv.at[1], isem.at[1])
        pltpu.async_copy(t_ref.at[srcv.at[0, 0]], bufs[0], gsem.at[0])

        @pl.loop(0, CPW_A // 16)
        def _(v):
            for jj in range(16):
                j = 16 * v + jj
                b, bn = jj % 2, 1 - jj % 2
                sb, l = (jj // 8) % 2, jj % 8
                sbn, ln = ((jj + 1) // 8) % 2, (jj + 1) % 8

                if jj == 0:
                    @pl.when(v > 0)
                    def _(v=v):
                        pltpu.async_copy(
                            src_ref.at[w, pl.ds(pl.multiple_of(
                                8 * (2 * v + 1), 8), 8)],
                            srcv.at[1], isem.at[1])
                if jj == 8:
                    @pl.when(v < CPW_A // 16 - 1)
                    def _(v=v):
                        pltpu.async_copy(
                            src_ref.at[w, pl.ds(pl.multiple_of(
                                8 * (2 * v + 2), 8), 8)],
                            srcv.at[0], isem.at[0])

                pltpu.make_async_copy(t_ref.at[srcv.at[sb, l]], bufs[b],
                                      gsem.at[b]).wait()

                @pl.when(j + 1 < CPW_A)
                def _(b=b, bn=bn, sbn=sbn, ln=ln, jj=jj):
                    if jj in (7, 15):
                        pltpu.make_async_copy(
                            src_ref.at[w, pl.ds(0, 8)], srcv.at[sbn],
                            isem.at[sbn]).wait()
                    pltpu.async_copy(t_ref.at[srcv.at[sbn, ln]], bufs[bn],
                                     gsem.at[bn])

                pltpu.sync_copy(bufs[b], acc.at[didx.at[j]], add=True)

        plsc.subcore_barrier()
        pltpu.sync_copy(acc.at[rs], out_ref.at[cid, rs])

        @pl.when(sid == NS - 1)
        def _():
            pltpu.sync_copy(acc.at[ts], out_ref.at[cid, ts])

    return k(t, src_w, dst_w)


def _elu(v):
    return jnp.where(v > 0, v, jnp.exp(jnp.where(v > 0, 0.0, v)) - 1.0)


def _dot(a, b):
    return jnp.dot(a, b, preferred_element_type=jnp.float32,
                   precision=lax.Precision.HIGHEST)


def _tc_layer1(x, w1, cnt):
    def body(x_ref, w_ref, ca_ref, cb_ref, t_ref, dinv_ref):
        deg = ca_ref[0, :, 0:1] + cb_ref[0, :, 0:1] - 1.0
        dinv = lax.rsqrt(deg)
        dinv_ref[...] = dinv
        t_ref[...] = _dot(x_ref[...], w_ref[...]) * dinv

    return pl.pallas_call(
        body,
        grid=(N // RB,),
        in_specs=[
            pl.BlockSpec((RB, D), lambda i: (i, 0)),
            pl.BlockSpec((D, D), lambda i: (0, 0)),
            pl.BlockSpec((1, RB, DEGW), lambda i: (0, i, 0)),
            pl.BlockSpec((1, RB, DEGW), lambda i: (1, i, 0)),
        ],
        out_specs=[
            pl.BlockSpec((RB, D), lambda i: (i, 0)),
            pl.BlockSpec((RB, 1), lambda i: (i, 0)),
        ],
        out_shape=[
            jax.ShapeDtypeStruct((N, D), jnp.float32),
            jax.ShapeDtypeStruct((N, 1), jnp.float32),
        ],
    )(x, w1, cnt, cnt)


def _tc_layer_next(part, t_prev, dinv, b_prev, w_next):
    def body(pa_ref, pb_ref, tp_ref, dinv_ref, b_ref, w_ref, t_ref):
        dinv = dinv_ref[...]
        agg = pa_ref[0] + pb_ref[0] - tp_ref[...]
        h = _elu(dinv * agg + b_ref[...])
        t_ref[...] = _dot(h, w_ref[...]) * dinv

    return pl.pallas_call(
        body,
        grid=(N // RB,),
        in_specs=[
            pl.BlockSpec((1, RB, D), lambda i: (0, i, 0)),
            pl.BlockSpec((1, RB, D), lambda i: (1, i, 0)),
            pl.BlockSpec((RB, D), lambda i: (i, 0)),
            pl.BlockSpec((RB, 1), lambda i: (i, 0)),
            pl.BlockSpec((1, D), lambda i: (0, 0)),
            pl.BlockSpec((D, D), lambda i: (0, 0)),
        ],
        out_specs=pl.BlockSpec((RB, D), lambda i: (i, 0)),
        out_shape=jax.ShapeDtypeStruct((N, D), jnp.float32),
    )(part, part, t_prev, dinv, b_prev, w_next)


def _tc_offsets(batch_col):
    def body(b_ref, o_ref):
        hist = jnp.zeros((1, G), jnp.float32)
        for i in range(N // RB):
            vals = b_ref[i * RB:(i + 1) * RB, :]
            eq = (vals == lax.broadcasted_iota(jnp.int32, (RB, G), 1))
            hist = hist + jnp.sum(eq.astype(jnp.float32), axis=0, keepdims=True)
        row = lax.broadcasted_iota(jnp.int32, (G, G), 0)
        col = lax.broadcasted_iota(jnp.int32, (G, G), 1)
        strict_lower = (row < col).astype(jnp.float32)
        starts = _dot(hist, strict_lower)
        o_ref[...] = starts.astype(jnp.int32)

    return pl.pallas_call(
        body,
        in_specs=[pl.BlockSpec((N, 1), lambda: (0, 0))],
        out_specs=pl.BlockSpec((1, G), lambda: (0, 0)),
        out_shape=jax.ShapeDtypeStruct((1, G), jnp.int32),
    )(batch_col)


def _tc_h3(part3, t3, dinv, b3):
    def body(pa_ref, pb_ref, tp_ref, dinv_ref, b_ref, h_ref):
        agg = pa_ref[0] + pb_ref[0] - tp_ref[...]
        h = _elu(dinv_ref[...] * agg + b_ref[...])
        rid = (pl.program_id(0) * RB_H
               + lax.broadcasted_iota(jnp.int32, (RB_H, D), 0))
        h_ref[...] = jnp.where(rid < N, h, -1e30)

    return pl.pallas_call(
        body,
        grid=(N_H // RB_H,),
        in_specs=[
            pl.BlockSpec((1, RB_H, D), lambda i: (0, i, 0)),
            pl.BlockSpec((1, RB_H, D), lambda i: (1, i, 0)),
            pl.BlockSpec((RB_H, D), lambda i: (i, 0)),
            pl.BlockSpec((RB_H, 1), lambda i: (i, 0)),
            pl.BlockSpec((1, D), lambda i: (0, 0)),
        ],
        out_specs=pl.BlockSpec((RB_H, D), lambda i: (i, 0)),
        out_shape=jax.ShapeDtypeStruct((N_H, D), jnp.float32),
    )(part3, part3, t3, dinv, b3)


def _sc_pool(h3, starts_ext):
    @functools.partial(
        pl.kernel,
        out_type=jax.ShapeDtypeStruct((NW, 8, D), jnp.float32),
        mesh=_mesh(),
        scratch_types=[
            pltpu.VMEM((G + 32,), jnp.int32),
            pltpu.VMEM((CB, D), jnp.float32),
            pltpu.VMEM((8, D), jnp.float32),
        ],
    )
    def k(h_ref, st_ref, out_ref, smem, buf, vout):
        cid = lax.axis_index("c")
        sid = lax.axis_index("s")
        w = cid * NS + sid
        pltpu.sync_copy(st_ref, smem)
        neg = jnp.full((16,), -1e30, jnp.float32)
        for gi in range(8):
            for f in range(8):
                vout[gi, pl.ds(16 * f, 16)] = neg
        for gi in range(GPT):
            g = w * GPT + gi
            sv = smem[pl.ds(g, 16)]
            s_g = sv[0]
            e_g = sv[1]
            c0 = (s_g // 8) * 8
            nch = (e_g - c0 + CB - 1) // CB

            def chunk_body(c, accs, c0=c0, s_g=s_g, e_g=e_g):
                base = pl.multiple_of(c0 + c * CB, 8)
                pltpu.sync_copy(h_ref.at[pl.ds(base, CB)], buf)
                rlo = jnp.maximum(base, s_g)
                rhi = jnp.minimum(base + CB, e_g)

                def row_body(r, a, base=base):
                    loc = r - base
                    return tuple(
                        jnp.maximum(a[f], buf[loc, pl.ds(16 * f, 16)])
                        for f in range(8))

                return lax.fori_loop(rlo, rhi, row_body, accs)

            accs = lax.fori_loop(0, nch, chunk_body, (neg,) * 8)
            for f in range(8):
                vout[gi, pl.ds(16 * f, 16)] = accs[f]
        pltpu.sync_copy(vout, out_ref.at[w])

    return k(h3, starts_ext)


def _tc_head(pooled_flat, wo, bo):
    def body(p_ref, wo_ref, bo_ref, o_ref):
        row = lax.broadcasted_iota(jnp.int32, (G, NW * 8), 0)
        col = lax.broadcasted_iota(jnp.int32, (G, NW * 8), 1)
        sel = (col == 8 * (row // GPT) + row % GPT).astype(jnp.float32)
        pooled = _dot(sel, p_ref[...])
        pooled = jnp.where(pooled < -1e29, 0.0, pooled)
        logits = _dot(pooled, wo_ref[...]) + bo_ref[...]
        m = jnp.max(logits, axis=1, keepdims=True)
        ex = jnp.exp(logits - m)
        o_ref[...] = ex / jnp.sum(ex, axis=1, keepdims=True)

    return pl.pallas_call(
        body,
        in_specs=[
            pl.BlockSpec((NW * 8, D), lambda: (0, 0)),
            pl.BlockSpec((D, NCLS), lambda: (0, 0)),
            pl.BlockSpec((1, NCLS), lambda: (0, 0)),
        ],
        out_specs=pl.BlockSpec((G, NCLS), lambda: (0, 0)),
        out_shape=jax.ShapeDtypeStruct((G, NCLS), jnp.float32),
    )(pooled_flat, wo, bo)


def kernel(x, edge_index, batch, W1, b1, W2, b2, W3, b3, Wo, bo):
    src32 = edge_index[0].astype(jnp.int32).reshape(NW, EPW)
    dst32 = edge_index[1].astype(jnp.int32).reshape(NW, EPW)
    src_w = src32.reshape(NW, CPW_A, CHUNK_A)
    dst_w = dst32.reshape(NW, CPW_A, CHUNK_A)
    dst_deg = jnp.pad(dst32, ((0, 0), (0, PAD)),
                      constant_values=N).reshape(NW, CPW, CHUNK)
    batch_col = batch.astype(jnp.int32).reshape(N, 1)
    ones_hbm = jnp.ones((RPT, DEGW), jnp.float32)

    cnt = _sc_degree(ones_hbm, dst_deg)
    t1, dinv = _tc_layer1(x, W1, cnt)
    p1 = _sc_aggregate(t1, src_w, dst_w)
    t2 = _tc_layer_next(p1, t1, dinv, b1.reshape(1, D), W2)
    p2 = _sc_aggregate(t2, src_w, dst_w)
    t3 = _tc_layer_next(p2, t2, dinv, b2.reshape(1, D), W3)
    p3 = _sc_aggregate(t3, src_w, dst_w)
    starts = _tc_offsets(batch_col)
    h3 = _tc_h3(p3, t3, dinv, b3.reshape(1, D))
    starts_ext = jnp.concatenate(
        [starts.reshape(G), jnp.full((32,), N, jnp.int32)])
    pooled_b = _sc_pool(h3, starts_ext)
    return _tc_head(pooled_b.reshape(NW * 8, D), Wo, bo.reshape(1, NCLS))
